# baseline (speedup 1.0000x reference)
import functools
import math

import jax
import jax.numpy as jnp
from jax import lax
from jax.experimental import pallas as pl
from jax.experimental.pallas import tpu as pltpu
from jax.experimental.pallas import tpu_sc as plsc


@functools.lru_cache(maxsize=None)
def _build(B, L, D, V, maxlen):
    info = plsc.get_sparse_core_info()
    NC, NS, LANES = info.num_cores, info.num_subcores, info.num_lanes
    NW = NC * NS
    assert B % NW == 0 and V % (2 * NW) == 0 and D % LANES == 0
    scale = math.sqrt(D)
    NJ = D // LANES

    mesh = plsc.VectorSubcoreMesh(core_axis_name="c", subcore_axis_name="s")

    seqs_w = B // NW
    G = 2
    n_chunks_b = seqs_w // G
    ROWS = G * L

    @functools.partial(
        pl.kernel,
        out_type=jax.ShapeDtypeStruct((B * L, 2 * D), jnp.float32),
        mesh=mesh,
        compiler_params=pltpu.CompilerParams(use_tc_tiling_on_sc=False),
        scratch_types=[
            pltpu.VMEM((ROWS,), jnp.int32),
            pltpu.VMEM((ROWS,), jnp.int32),
            pltpu.VMEM((ROWS, D), jnp.float32),
            pltpu.VMEM((ROWS, D), jnp.float32),
            pltpu.VMEM((L, D), jnp.float32),
            pltpu.SemaphoreType.DMA,
            pltpu.SemaphoreType.DMA,
            pltpu.SemaphoreType.DMA,
            pltpu.SemaphoreType.DMA,
        ],
    )
    def gather(
        x_hbm, tok_hbm, pos_hbm, out_hbm,
        idx0, idx1, rows0, rows1, pos_v, gs0, gs1, os0, os1,
    ):
        wid = lax.axis_index("s") * NC + lax.axis_index("c")
        idxs = (idx0, idx1)
        rows = (rows0, rows1)
        gsems = (gs0, gs1)
        osems = (os0, os1)
        pltpu.sync_copy(pos_hbm.at[pl.ds(0, L)], pos_v)

        def fetch(c, sl):
            base = (wid * n_chunks_b + c) * ROWS
            base = pl.multiple_of(base, 8)
            pltpu.sync_copy(x_hbm.at[pl.ds(base, ROWS)], idxs[sl])
            pltpu.async_copy(tok_hbm.at[idxs[sl]], rows[sl], gsems[sl])

        fetch(0, 0)

        def chunk_body(c, carry):
            base = (wid * n_chunks_b + c) * ROWS
            base = pl.multiple_of(base, 8)

            for sl in range(2):

                @pl.when((c % 2) == sl)
                def _():
                    rows_v = rows[sl]
                    @pl.when(c + 1 < n_chunks_b)
                    def _():
                        @pl.when(c >= 1)
                        def _():
                            prev = (wid * n_chunks_b + c - 1) * ROWS
                            prev = pl.multiple_of(prev, 8)
                            pltpu.make_async_copy(
                                rows[1 - sl],
                                out_hbm.at[pl.ds(prev, ROWS), pl.ds(0, D)],
                                osems[1 - sl],
                            ).wait()

                        fetch(c + 1, 1 - sl)

                    pltpu.make_async_copy(
                        tok_hbm.at[idxs[sl]], rows_v, gsems[sl]
                    ).wait()

                    def l_body(l, lc):
                        p = [
                            pos_v[l, pl.ds(LANES * j, LANES)]
                            for j in range(NJ)
                        ]
                        for g in range(G):
                            r = g * L + l
                            for j in range(NJ):
                                rows_v[r, pl.ds(LANES * j, LANES)] = (
                                    rows_v[r, pl.ds(LANES * j, LANES)] + p[j]
                                ) * scale
                        return lc

                    lax.fori_loop(0, L, l_body, 0)
                    pltpu.async_copy(
                        rows_v,
                        out_hbm.at[pl.ds(base, ROWS), pl.ds(0, D)],
                        osems[sl],
                    )

            return carry

        lax.fori_loop(0, n_chunks_b, chunk_body, 0)
        for sl in range(2):
            c_last = n_chunks_b - 2 + sl
            base = (wid * n_chunks_b + c_last) * ROWS
            base = pl.multiple_of(base, 8)
            pltpu.make_async_copy(
                rows[c_last % 2],
                out_hbm.at[pl.ds(base, ROWS), pl.ds(0, D)],
                osems[c_last % 2],
            ).wait()

    GC = 1
    n_chunks_c = seqs_w // GC
    CR = GC * L

    @functools.partial(
        pl.kernel,
        out_type=jax.ShapeDtypeStruct((B, L, D), jnp.float32),
        mesh=mesh,
        scratch_types=[
            pltpu.VMEM((CR, 2 * D), jnp.float32),
            pltpu.VMEM((CR, 2 * D), jnp.float32),
            pltpu.VMEM((CR, D), jnp.float32),
            pltpu.SemaphoreType.DMA,
            pltpu.SemaphoreType.DMA,
            pltpu.SemaphoreType.DMA,
        ],
    )
    def repack(wide_hbm, out_hbm, b0, b1, bn, s0, s1, ws):
        wid = lax.axis_index("s") * NC + lax.axis_index("c")
        bufs = (b0, b1)
        sems = (s0, s1)

        def read(c, sl):
            r0 = (wid * n_chunks_c + c) * CR
            pltpu.async_copy(wide_hbm.at[pl.ds(r0, CR)], bufs[sl], sems[sl])

        read(0, 0)

        def chunk_body(c, carry):
            seq0 = (wid * n_chunks_c + c) * GC
            r0 = seq0 * L

            for sl in range(2):

                @pl.when((c % 2) == sl)
                def _():
                    pltpu.make_async_copy(
                        wide_hbm.at[pl.ds(r0, CR)], bufs[sl], sems[sl]
                    ).wait()

                    @pl.when(c + 1 < n_chunks_c)
                    def _():
                        read(c + 1, 1 - sl)

                    @pl.when(c >= 1)
                    def _():
                        pltpu.make_async_copy(
                            bn, out_hbm.at[seq0], ws
                        ).wait()

                    def row_body(r, rc):
                        for j in range(NJ):
                            bn[r, pl.ds(j * LANES, LANES)] = bufs[sl][
                                r, pl.ds(j * LANES, LANES)
                            ]
                        return rc

                    lax.fori_loop(0, CR, row_body, 0)
                    pltpu.async_copy(bn, out_hbm.at[seq0], ws)

            return carry

        lax.fori_loop(0, n_chunks_c, chunk_body, 0)
        pltpu.make_async_copy(bn, out_hbm.at[0], ws).wait()

    def run(x, token_table, pos_table):
        x_flat = x.reshape(B * L).astype(jnp.int32)
        gathered = gather(x_flat, token_table, pos_table)
        return repack(gathered)

    return run


def kernel(x, token_table, pos_table):
    B, L = x.shape
    V, D = token_table.shape
    run = _build(B, L, D, V, pos_table.shape[0])
    return run(x, token_table, pos_table)


# speedup vs baseline: 1.4612x; 1.4612x over previous
import functools
import math

import jax
import jax.numpy as jnp
from jax import lax
from jax.experimental import pallas as pl
from jax.experimental.pallas import tpu as pltpu
from jax.experimental.pallas import tpu_sc as plsc


@functools.lru_cache(maxsize=None)
def _build(B, L, D, V, maxlen):
    info = plsc.get_sparse_core_info()
    NC, NS, LANES = info.num_cores, info.num_subcores, info.num_lanes
    NW = NC * NS
    assert B % NW == 0 and V % (2 * NW) == 0 and D % LANES == 0
    scale = math.sqrt(D)
    NJ = D // LANES

    mesh = plsc.VectorSubcoreMesh(core_axis_name="c", subcore_axis_name="s")

    seqs_w = B // NW
    G = 4
    n_chunks_b = seqs_w // G
    ROWS = G * L

    @functools.partial(
        pl.kernel,
        out_type=jax.ShapeDtypeStruct((B * L, 2 * D), jnp.float32),
        mesh=mesh,
        compiler_params=pltpu.CompilerParams(use_tc_tiling_on_sc=False),
        scratch_types=[
            pltpu.VMEM((ROWS,), jnp.int32),
            pltpu.VMEM((ROWS,), jnp.int32),
            pltpu.VMEM((ROWS, D), jnp.float32),
            pltpu.VMEM((ROWS, D), jnp.float32),
            pltpu.VMEM((L, D), jnp.float32),
            pltpu.SemaphoreType.DMA,
            pltpu.SemaphoreType.DMA,
            pltpu.SemaphoreType.DMA,
            pltpu.SemaphoreType.DMA,
        ],
    )
    def gather(
        x_hbm, tok_hbm, pos_hbm, out_hbm,
        idx0, idx1, rows0, rows1, pos_v, gs0, gs1, os0, os1,
    ):
        wid = lax.axis_index("s") * NC + lax.axis_index("c")
        idxs = (idx0, idx1)
        rows = (rows0, rows1)
        gsems = (gs0, gs1)
        osems = (os0, os1)
        pltpu.sync_copy(pos_hbm.at[pl.ds(0, L)], pos_v)

        def fetch(c, sl):
            base = (wid * n_chunks_b + c) * ROWS
            base = pl.multiple_of(base, 8)
            pltpu.sync_copy(x_hbm.at[pl.ds(base, ROWS)], idxs[sl])
            pltpu.async_copy(tok_hbm.at[idxs[sl]], rows[sl], gsems[sl])

        fetch(0, 0)

        def chunk_body(c, carry):
            base = (wid * n_chunks_b + c) * ROWS
            base = pl.multiple_of(base, 8)

            for sl in range(2):

                @pl.when((c % 2) == sl)
                def _():
                    rows_v = rows[sl]
                    @pl.when(c + 1 < n_chunks_b)
                    def _():
                        @pl.when(c >= 1)
                        def _():
                            prev = (wid * n_chunks_b + c - 1) * ROWS
                            prev = pl.multiple_of(prev, 8)
                            pltpu.make_async_copy(
                                rows[1 - sl],
                                out_hbm.at[pl.ds(prev, ROWS), pl.ds(0, D)],
                                osems[1 - sl],
                            ).wait()

                        fetch(c + 1, 1 - sl)

                    pltpu.make_async_copy(
                        tok_hbm.at[idxs[sl]], rows_v, gsems[sl]
                    ).wait()

                    def l_body(l, lc):
                        p = [
                            pos_v[l, pl.ds(LANES * j, LANES)]
                            for j in range(NJ)
                        ]
                        for g in range(G):
                            r = g * L + l
                            for j in range(NJ):
                                rows_v[r, pl.ds(LANES * j, LANES)] = (
                                    rows_v[r, pl.ds(LANES * j, LANES)] + p[j]
                                ) * scale
                        return lc

                    lax.fori_loop(0, L, l_body, 0)
                    pltpu.async_copy(
                        rows_v,
                        out_hbm.at[pl.ds(base, ROWS), pl.ds(0, D)],
                        osems[sl],
                    )

            return carry

        lax.fori_loop(0, n_chunks_b, chunk_body, 0)
        for sl in range(2):
            c_last = n_chunks_b - 2 + sl
            base = (wid * n_chunks_b + c_last) * ROWS
            base = pl.multiple_of(base, 8)
            pltpu.make_async_copy(
                rows[c_last % 2],
                out_hbm.at[pl.ds(base, ROWS), pl.ds(0, D)],
                osems[c_last % 2],
            ).wait()

    def run(x, token_table, pos_table):
        x_flat = x.reshape(B * L).astype(jnp.int32)
        gathered = gather(x_flat, token_table, pos_table)
        return gathered.reshape(B, L, 2 * D)[:, :, :D]

    return run


def kernel(x, token_table, pos_table):
    B, L = x.shape
    V, D = token_table.shape
    run = _build(B, L, D, V, pos_table.shape[0])
    return run(x, token_table, pos_table)
